# Initial kernel scaffold; baseline (speedup 1.0000x reference)
import jax
import jax.numpy as jnp
from jax.experimental import pallas as pl


def kernel(pred0, pred1, pred2, pred3, pred4, imgs, mask, pw0, pb0, pw1, pb1, pw2, pb2, pw3, pb3, pw4, pb4):
    raise NotImplementedError("write your pallas kernel here")


# speedup vs baseline: 54.9589x; 54.9589x over previous
import math

import jax
import jax.numpy as jnp
from jax.experimental import pallas as pl
import jax.experimental.pallas.tpu as pltpu

_POOLS = (16, 8, 4, 2, 1)
_NB = 9


def _loss_kernel(params_ref, xp_ref, mask_ref, p0_ref, p1_ref, p2_ref,
                 p3_ref, p4_ref, out_ref):
    H = mask_ref.shape[2]
    W = mask_ref.shape[3]
    f32 = jnp.float32

    g0s = []
    g1s = []
    for c in range(3):
        x = xp_ref[0, c, :, :]
        colsum = x[0:H, :] + 2.0 * x[1:H + 1, :] + x[2:H + 2, :]
        g0 = colsum[:, 0:W] - colsum[:, 2:W + 2]
        rowsum = x[:, 0:W] + 2.0 * x[:, 1:W + 1] + x[:, 2:W + 2]
        g1 = rowsum[0:H, :] - rowsum[2:H + 2, :]
        g0s.append(g0)
        g1s.append(g1)

    mask2d = mask_ref[0, 0, :, :]
    masksum = jnp.sum(mask2d)

    bnd = [(math.cos(k * math.pi / _NB), math.sin(k * math.pi / _NB))
           for k in range(1, _NB)]

    pred_refs = {16: p0_ref, 8: p1_ref, 4: p2_ref, 2: p3_ref, 1: p4_ref}
    nums = []
    for k, s in enumerate(_POOLS):
        Hs, Ws = H // s, W // s
        pred_ref = pred_refs[s]
        if s == 1:
            Mk = mask2d
        else:
            rp = (jax.lax.broadcasted_iota(jnp.int32, (Hs, H), 1) // s ==
                  jax.lax.broadcasted_iota(jnp.int32, (Hs, H), 0)).astype(f32)
            cp = (jax.lax.broadcasted_iota(jnp.int32, (W, Ws), 0) // s ==
                  jax.lax.broadcasted_iota(jnp.int32, (W, Ws), 1)).astype(f32)
            Mk = jnp.dot(jnp.dot(rp, mask2d, preferred_element_type=f32), cp,
                         preferred_element_type=f32) * (1.0 / (s * s))
            rs = (jax.lax.broadcasted_iota(jnp.int32, (Hs, H), 1) ==
                  s * jax.lax.broadcasted_iota(jnp.int32, (Hs, H), 0)
                  ).astype(f32)
            cs = (jax.lax.broadcasted_iota(jnp.int32, (W, Ws), 0) ==
                  s * jax.lax.broadcasted_iota(jnp.int32, (W, Ws), 1)
                  ).astype(f32)

        e = jnp.zeros((Hs, Ws), dtype=f32)
        for c in range(3):
            if s == 1:
                gg0, gg1 = g0s[c], g1s[c]
            else:
                gg0 = jnp.dot(jnp.dot(rs, g0s[c], preferred_element_type=f32),
                              cs, preferred_element_type=f32)
                gg1 = jnp.dot(jnp.dot(rs, g1s[c], preferred_element_type=f32),
                              cs, preferred_element_type=f32)
            mag = jnp.sqrt(gg0 * gg0 + gg1 * gg1)
            neg = gg0 < 0.0
            a0 = jnp.where(neg, -gg0, gg0)
            a1 = jnp.where(neg, -gg1, gg1)
            tests = [a0 * cb - a1 * sb >= 0.0 for (cb, sb) in bnd]
            wrap = jnp.logical_and(a0 == 0.0, a1 < 0.0)
            mbs = []
            for b in range(_NB):
                if b == 0:
                    oh = jnp.logical_or(jnp.logical_not(tests[0]), wrap)
                elif b == _NB - 1:
                    oh = jnp.logical_and(tests[_NB - 2],
                                         jnp.logical_not(wrap))
                else:
                    oh = jnp.logical_and(tests[b - 1],
                                         jnp.logical_not(tests[b]))
                mbs.append(jnp.where(oh, mag, 0.0))
            base = k * (_NB * _NB + _NB)
            hogs = []
            for o in range(_NB):
                h = params_ref[base + _NB * _NB + o]
                for b in range(_NB):
                    h = h + params_ref[base + o * _NB + b] * mbs[b]
                hogs.append(h)
            nrm2 = hogs[0] * hogs[0]
            for o in range(1, _NB):
                nrm2 = nrm2 + hogs[o] * hogs[o]
            inv = 1.0 / jnp.maximum(jnp.sqrt(nrm2), 1e-12)
            for o in range(_NB):
                d = pred_ref[0, c * _NB + o, :, :] - hogs[o] * inv
                e = e + d * d
        nums.append(jnp.sum(e * Mk) * (1.0 / 27.0))

    lane = jax.lax.broadcasted_iota(jnp.int32, (8, 128), 1)
    tile = jnp.zeros((8, 128), dtype=f32)
    for k in range(5):
        tile = jnp.where(lane == k, nums[k], tile)
    tile = jnp.where(lane == 5, masksum, tile)
    out_ref[0, :, :] = tile


def kernel(pred0, pred1, pred2, pred3, pred4, imgs, mask, pw0, pb0, pw1, pb1,
           pw2, pb2, pw3, pb3, pw4, pb4):
    N, _, H, W = imgs.shape
    xp = jnp.pad(imgs, ((0, 0), (0, 0), (1, 1), (1, 1)), mode='reflect')
    params = jnp.concatenate([
        jnp.concatenate([pw.reshape(-1), pb])
        for pw, pb in ((pw0, pb0), (pw1, pb1), (pw2, pb2), (pw3, pb3),
                       (pw4, pb4))
    ])

    preds = (pred0, pred1, pred2, pred3, pred4)
    in_specs = [
        pl.BlockSpec(memory_space=pltpu.SMEM),
        pl.BlockSpec((1, 3, H + 2, W + 2), lambda n: (n, 0, 0, 0)),
        pl.BlockSpec((1, 1, H, W), lambda n: (n, 0, 0, 0)),
    ] + [
        pl.BlockSpec((1, 27, H // s, W // s), lambda n: (n, 0, 0, 0))
        for s in _POOLS
    ]
    out = pl.pallas_call(
        _loss_kernel,
        grid=(N,),
        in_specs=in_specs,
        out_specs=pl.BlockSpec((1, 8, 128), lambda n: (n, 0, 0)),
        out_shape=jax.ShapeDtypeStruct((N, 8, 128), jnp.float32),
        compiler_params=pltpu.CompilerParams(
            dimension_semantics=("arbitrary",)),
    )(params, xp, mask, *preds)

    vals = jnp.sum(out[:, 0, :], axis=0)
    masksum = vals[5]
    loss = jnp.float32(0.0)
    for k, s in enumerate(_POOLS):
        loss = loss + vals[k] * (s * s) / masksum
    return loss


# speedup vs baseline: 66.3471x; 1.2072x over previous
import math

import jax
import jax.numpy as jnp
from jax.experimental import pallas as pl
import jax.experimental.pallas.tpu as pltpu

_POOLS = (16, 8, 4, 2, 1)
_NB = 9


def _loss_kernel(params_ref, xp_ref, mask_ref, p0_ref, p1_ref, p2_ref,
                 p3_ref, p4_ref, out_ref):
    H = mask_ref.shape[2]
    W = mask_ref.shape[3]
    f32 = jnp.float32

    g0s = []
    g1s = []
    for c in range(3):
        x = xp_ref[0, c, :, :]
        colsum = x[0:H, :] + 2.0 * x[1:H + 1, :] + x[2:H + 2, :]
        g0 = colsum[:, 0:W] - colsum[:, 2:W + 2]
        rowsum = x[:, 0:W] + 2.0 * x[:, 1:W + 1] + x[:, 2:W + 2]
        g1 = rowsum[0:H, :] - rowsum[2:H + 2, :]
        g0s.append(g0)
        g1s.append(g1)

    mask2d = mask_ref[0, 0, :, :]
    masksum = jnp.sum(mask2d)

    cots = [1.0 / math.tan(k * math.pi / _NB) for k in range(1, _NB)]

    pred_refs = {16: p0_ref, 8: p1_ref, 4: p2_ref, 2: p3_ref, 1: p4_ref}
    nums = []
    for k, s in enumerate(_POOLS):
        Hs, Ws = H // s, W // s
        pred_ref = pred_refs[s]
        if s == 1:
            Mk = mask2d
        else:
            rp = (jax.lax.broadcasted_iota(jnp.int32, (Hs, H), 1) // s ==
                  jax.lax.broadcasted_iota(jnp.int32, (Hs, H), 0)).astype(f32)
            cp = (jax.lax.broadcasted_iota(jnp.int32, (W, Ws), 0) // s ==
                  jax.lax.broadcasted_iota(jnp.int32, (W, Ws), 1)).astype(f32)
            Mk = jnp.dot(jnp.dot(rp, mask2d, preferred_element_type=f32), cp,
                         preferred_element_type=f32) * (1.0 / (s * s))
            rs = (jax.lax.broadcasted_iota(jnp.int32, (Hs, H), 1) ==
                  s * jax.lax.broadcasted_iota(jnp.int32, (Hs, H), 0)
                  ).astype(f32)
            cs = (jax.lax.broadcasted_iota(jnp.int32, (W, Ws), 0) ==
                  s * jax.lax.broadcasted_iota(jnp.int32, (W, Ws), 1)
                  ).astype(f32)

        e = jnp.zeros((Hs, Ws), dtype=f32)
        for c in range(3):
            if s == 1:
                gg0, gg1 = g0s[c], g1s[c]
            else:
                gg0 = jnp.dot(jnp.dot(rs, g0s[c], preferred_element_type=f32),
                              cs, preferred_element_type=f32)
                gg1 = jnp.dot(jnp.dot(rs, g1s[c], preferred_element_type=f32),
                              cs, preferred_element_type=f32)
            mag = jnp.sqrt(gg0 * gg0 + gg1 * gg1)
            neg = gg0 < 0.0
            a0 = jnp.where(neg, -gg0, gg0)
            a1 = jnp.where(neg, -gg1, gg1)
            a1 = jnp.where(a0 == 0.0, jnp.abs(a1), a1)
            t = [None] + [a1 <= a0 * ck for ck in cots]
            base = k * (_NB * _NB + _NB)
            hogs = []
            for o in range(_NB):
                w = [params_ref[base + o * _NB + b] for b in range(_NB)]
                lo = jnp.where(t[2], jnp.where(t[3], w[3], w[2]),
                               jnp.where(t[1], w[1], w[0]))
                hi = jnp.where(t[6],
                               jnp.where(t[7], jnp.where(t[8], w[8], w[7]),
                                         w[6]),
                               jnp.where(t[5], w[5], w[4]))
                pwsel = jnp.where(t[4], hi, lo)
                hogs.append(params_ref[base + _NB * _NB + o] + mag * pwsel)
            nrm2 = hogs[0] * hogs[0]
            for o in range(1, _NB):
                nrm2 = nrm2 + hogs[o] * hogs[o]
            inv = 1.0 / jnp.maximum(jnp.sqrt(nrm2), 1e-12)
            for o in range(_NB):
                d = pred_ref[0, c * _NB + o, :, :] - hogs[o] * inv
                e = e + d * d
        nums.append(jnp.sum(e * Mk) * (1.0 / 27.0))

    lane = jax.lax.broadcasted_iota(jnp.int32, (8, 128), 1)
    tile = jnp.zeros((8, 128), dtype=f32)
    for k in range(5):
        tile = jnp.where(lane == k, nums[k], tile)
    tile = jnp.where(lane == 5, masksum, tile)
    out_ref[0, :, :] = tile


def kernel(pred0, pred1, pred2, pred3, pred4, imgs, mask, pw0, pb0, pw1, pb1,
           pw2, pb2, pw3, pb3, pw4, pb4):
    N, _, H, W = imgs.shape
    xp = jnp.pad(imgs, ((0, 0), (0, 0), (1, 1), (1, 1)), mode='reflect')
    params = jnp.concatenate([
        jnp.concatenate([pw.reshape(-1), pb])
        for pw, pb in ((pw0, pb0), (pw1, pb1), (pw2, pb2), (pw3, pb3),
                       (pw4, pb4))
    ])

    preds = (pred0, pred1, pred2, pred3, pred4)
    in_specs = [
        pl.BlockSpec(memory_space=pltpu.SMEM),
        pl.BlockSpec((1, 3, H + 2, W + 2), lambda n: (n, 0, 0, 0)),
        pl.BlockSpec((1, 1, H, W), lambda n: (n, 0, 0, 0)),
    ] + [
        pl.BlockSpec((1, 27, H // s, W // s), lambda n: (n, 0, 0, 0))
        for s in _POOLS
    ]
    out = pl.pallas_call(
        _loss_kernel,
        grid=(N,),
        in_specs=in_specs,
        out_specs=pl.BlockSpec((1, 8, 128), lambda n: (n, 0, 0)),
        out_shape=jax.ShapeDtypeStruct((N, 8, 128), jnp.float32),
        compiler_params=pltpu.CompilerParams(
            dimension_semantics=("arbitrary",)),
    )(params, xp, mask, *preds)

    vals = jnp.sum(out[:, 0, :], axis=0)
    masksum = vals[5]
    loss = jnp.float32(0.0)
    for k, s in enumerate(_POOLS):
        loss = loss + vals[k] * (s * s) / masksum
    return loss


# speedup vs baseline: 66.3508x; 1.0001x over previous
import math

import jax
import jax.numpy as jnp
from jax.experimental import pallas as pl
import jax.experimental.pallas.tpu as pltpu

_POOLS = (16, 8, 4, 2, 1)
_NB = 9


def _loss_kernel(params_ref, xp_ref, mask_ref, p0_ref, p1_ref, p2_ref,
                 p3_ref, p4_ref, out_ref):
    H = mask_ref.shape[2]
    W = mask_ref.shape[3]
    f32 = jnp.float32

    g0s = []
    g1s = []
    for c in range(3):
        x = xp_ref[0, c, :, :]
        colsum = x[0:H, :] + 2.0 * x[1:H + 1, :] + x[2:H + 2, :]
        g0 = colsum[:, 0:W] - colsum[:, 2:W + 2]
        rowsum = x[:, 0:W] + 2.0 * x[:, 1:W + 1] + x[:, 2:W + 2]
        g1 = rowsum[0:H, :] - rowsum[2:H + 2, :]
        g0s.append(g0)
        g1s.append(g1)

    mask2d = mask_ref[0, 0, :, :]
    masksum = jnp.sum(mask2d)

    cots = [1.0 / math.tan(k * math.pi / _NB) for k in range(1, _NB)]

    pred_refs = {16: p0_ref, 8: p1_ref, 4: p2_ref, 2: p3_ref, 1: p4_ref}
    nums = []
    for k, s in enumerate(_POOLS):
        Hs, Ws = H // s, W // s
        pred_ref = pred_refs[s]
        if s == 1:
            Mk = mask2d
        else:
            rp = (jax.lax.broadcasted_iota(jnp.int32, (Hs, H), 1) // s ==
                  jax.lax.broadcasted_iota(jnp.int32, (Hs, H), 0)).astype(f32)
            cp = (jax.lax.broadcasted_iota(jnp.int32, (W, Ws), 0) // s ==
                  jax.lax.broadcasted_iota(jnp.int32, (W, Ws), 1)).astype(f32)
            Mk = jnp.dot(jnp.dot(rp, mask2d, preferred_element_type=f32), cp,
                         preferred_element_type=f32) * (1.0 / (s * s))
            rs = (jax.lax.broadcasted_iota(jnp.int32, (Hs, H), 1) ==
                  s * jax.lax.broadcasted_iota(jnp.int32, (Hs, H), 0)
                  ).astype(f32)
            cs = (jax.lax.broadcasted_iota(jnp.int32, (W, Ws), 0) ==
                  s * jax.lax.broadcasted_iota(jnp.int32, (W, Ws), 1)
                  ).astype(f32)

        e = jnp.zeros((Hs, Ws), dtype=f32)
        for c in range(3):
            if s == 1:
                gg0, gg1 = g0s[c], g1s[c]
            else:
                gg0 = jnp.dot(jnp.dot(rs, g0s[c], preferred_element_type=f32),
                              cs, preferred_element_type=f32)
                gg1 = jnp.dot(jnp.dot(rs, g1s[c], preferred_element_type=f32),
                              cs, preferred_element_type=f32)
            mag = jnp.sqrt(gg0 * gg0 + gg1 * gg1)
            neg = gg0 < 0.0
            a0 = jnp.where(neg, -gg0, gg0)
            a1 = jnp.where(neg, -gg1, gg1)
            a1 = jnp.where(a0 == 0.0, jnp.abs(a1), a1)
            t = [None] + [a1 <= a0 * ck for ck in cots]
            base = k * (_NB * _NB + _NB)
            hogs = []
            for o in range(_NB):
                w = [params_ref[base + o * _NB + b] for b in range(_NB)]
                lo = jnp.where(t[2], jnp.where(t[3], w[3], w[2]),
                               jnp.where(t[1], w[1], w[0]))
                hi = jnp.where(t[6],
                               jnp.where(t[7], jnp.where(t[8], w[8], w[7]),
                                         w[6]),
                               jnp.where(t[5], w[5], w[4]))
                pwsel = jnp.where(t[4], hi, lo)
                hogs.append(params_ref[base + _NB * _NB + o] + mag * pwsel)
            nrm2 = hogs[0] * hogs[0]
            for o in range(1, _NB):
                nrm2 = nrm2 + hogs[o] * hogs[o]
            inv = 1.0 / jnp.maximum(jnp.sqrt(nrm2), 1e-12)
            for o in range(_NB):
                d = pred_ref[0, c * _NB + o, :, :] - hogs[o] * inv
                e = e + d * d
        nums.append(jnp.sum(e * Mk) * (1.0 / 27.0))

    lane = jax.lax.broadcasted_iota(jnp.int32, (8, 128), 1)
    tile = jnp.zeros((8, 128), dtype=f32)
    for k in range(5):
        tile = jnp.where(lane == k, nums[k], tile)
    tile = jnp.where(lane == 5, masksum, tile)
    out_ref[0, :, :] = tile


def kernel(pred0, pred1, pred2, pred3, pred4, imgs, mask, pw0, pb0, pw1, pb1,
           pw2, pb2, pw3, pb3, pw4, pb4):
    N, _, H, W = imgs.shape
    xp = jnp.pad(imgs, ((0, 0), (0, 0), (1, 1), (1, 1)), mode='reflect')
    params = jnp.concatenate([
        jnp.concatenate([pw.reshape(-1), pb])
        for pw, pb in ((pw0, pb0), (pw1, pb1), (pw2, pb2), (pw3, pb3),
                       (pw4, pb4))
    ])

    preds = (pred0, pred1, pred2, pred3, pred4)
    in_specs = [
        pl.BlockSpec(memory_space=pltpu.SMEM),
        pl.BlockSpec((1, 3, H + 2, W + 2), lambda n: (n, 0, 0, 0)),
        pl.BlockSpec((1, 1, H, W), lambda n: (n, 0, 0, 0)),
    ] + [
        pl.BlockSpec((1, 27, H // s, W // s), lambda n: (n, 0, 0, 0))
        for s in _POOLS
    ]
    out = pl.pallas_call(
        _loss_kernel,
        grid=(N,),
        in_specs=in_specs,
        out_specs=pl.BlockSpec((1, 8, 128), lambda n: (n, 0, 0)),
        out_shape=jax.ShapeDtypeStruct((N, 8, 128), jnp.float32),
        compiler_params=pltpu.CompilerParams(
            dimension_semantics=("parallel",)),
    )(params, xp, mask, *preds)

    vals = jnp.sum(out[:, 0, :], axis=0)
    masksum = vals[5]
    loss = jnp.float32(0.0)
    for k, s in enumerate(_POOLS):
        loss = loss + vals[k] * (s * s) / masksum
    return loss


# speedup vs baseline: 77.3559x; 1.1659x over previous
import math

import jax
import jax.numpy as jnp
from jax.experimental import pallas as pl
import jax.experimental.pallas.tpu as pltpu

_POOLS = (16, 8, 4, 2, 1)
_NB = 9


def _tree(t, w):
    lo = jnp.where(t[2], jnp.where(t[3], w[3], w[2]),
                   jnp.where(t[1], w[1], w[0]))
    hi = jnp.where(t[6],
                   jnp.where(t[7], jnp.where(t[8], w[8], w[7]), w[6]),
                   jnp.where(t[5], w[5], w[4]))
    return jnp.where(t[4], hi, lo)


def _loss_kernel(params_ref, packed_ref, xp_ref, mask_ref, p0_ref, p1_ref,
                 p2_ref, p3_ref, p4_ref, out_ref):
    H = mask_ref.shape[2]
    W = mask_ref.shape[3]
    f32 = jnp.float32

    g0s = []
    g1s = []
    for c in range(3):
        x = xp_ref[0, c, :, :]
        colsum = x[0:H, :] + 2.0 * x[1:H + 1, :] + x[2:H + 2, :]
        g0 = colsum[:, 0:W] - colsum[:, 2:W + 2]
        rowsum = x[:, 0:W] + 2.0 * x[:, 1:W + 1] + x[:, 2:W + 2]
        g1 = rowsum[0:H, :] - rowsum[2:H + 2, :]
        g0s.append(g0)
        g1s.append(g1)

    mask2d = mask_ref[0, 0, :, :]
    masksum = jnp.sum(mask2d)

    cots = [1.0 / math.tan(k * math.pi / _NB) for k in range(1, _NB)]

    pred_refs = {16: p0_ref, 8: p1_ref, 4: p2_ref, 2: p3_ref, 1: p4_ref}
    nums = []
    for k, s in enumerate(_POOLS):
        Hs, Ws = H // s, W // s
        pred_ref = pred_refs[s]
        if s == 1:
            Mk = mask2d
        else:
            rp = (jax.lax.broadcasted_iota(jnp.int32, (Hs, H), 1) // s ==
                  jax.lax.broadcasted_iota(jnp.int32, (Hs, H), 0)).astype(f32)
            cp = (jax.lax.broadcasted_iota(jnp.int32, (W, Ws), 0) // s ==
                  jax.lax.broadcasted_iota(jnp.int32, (W, Ws), 1)).astype(f32)
            Mk = jnp.dot(jnp.dot(rp, mask2d, preferred_element_type=f32), cp,
                         preferred_element_type=f32) * (1.0 / (s * s))
            rs = (jax.lax.broadcasted_iota(jnp.int32, (Hs, H), 1) ==
                  s * jax.lax.broadcasted_iota(jnp.int32, (Hs, H), 0)
                  ).astype(f32)
            cs = (jax.lax.broadcasted_iota(jnp.int32, (W, Ws), 0) ==
                  s * jax.lax.broadcasted_iota(jnp.int32, (W, Ws), 1)
                  ).astype(f32)

        e = jnp.zeros((Hs, Ws), dtype=f32)
        for c in range(3):
            if s == 1:
                gg0, gg1 = g0s[c], g1s[c]
            else:
                gg0 = jnp.dot(jnp.dot(rs, g0s[c], preferred_element_type=f32),
                              cs, preferred_element_type=f32)
                gg1 = jnp.dot(jnp.dot(rs, g1s[c], preferred_element_type=f32),
                              cs, preferred_element_type=f32)
            mag = jnp.sqrt(gg0 * gg0 + gg1 * gg1)
            neg = gg0 < 0.0
            a0 = jnp.where(neg, -gg0, gg0)
            a1 = jnp.where(neg, -gg1, gg1)
            a1 = jnp.where(a0 == 0.0, jnp.abs(a1), a1)
            t = [None] + [a1 <= a0 * ck for ck in cots]
            base = k * (2 * _NB)
            baseu = k * (4 * _NB)
            hogs = [None] * _NB
            for j in range(4):
                w = [packed_ref[baseu + j * _NB + b] for b in range(_NB)]
                v = _tree(t, w)
                hi = jax.lax.bitcast_convert_type(
                    v & jnp.uint32(0xFFFF0000), jnp.float32)
                lo = jax.lax.bitcast_convert_type(v << 16, jnp.float32)
                hogs[2 * j] = params_ref[base + _NB + 2 * j] + mag * hi
                hogs[2 * j + 1] = params_ref[base + _NB + 2 * j + 1] + mag * lo
            w8 = [params_ref[base + b] for b in range(_NB)]
            hogs[8] = params_ref[base + _NB + 8] + mag * _tree(t, w8)
            nrm2 = hogs[0] * hogs[0]
            for o in range(1, _NB):
                nrm2 = nrm2 + hogs[o] * hogs[o]
            inv = 1.0 / jnp.maximum(jnp.sqrt(nrm2), 1e-12)
            for o in range(_NB):
                d = pred_ref[0, c * _NB + o, :, :] - hogs[o] * inv
                e = e + d * d
        nums.append(jnp.sum(e * Mk) * (1.0 / 27.0))

    lane = jax.lax.broadcasted_iota(jnp.int32, (8, 128), 1)
    tile = jnp.zeros((8, 128), dtype=f32)
    for k in range(5):
        tile = jnp.where(lane == k, nums[k], tile)
    tile = jnp.where(lane == 5, masksum, tile)
    out_ref[0, :, :] = tile


def kernel(pred0, pred1, pred2, pred3, pred4, imgs, mask, pw0, pb0, pw1, pb1,
           pw2, pb2, pw3, pb3, pw4, pb4):
    N, _, H, W = imgs.shape
    xp = jnp.pad(imgs, ((0, 0), (0, 0), (1, 1), (1, 1)), mode='reflect')
    pws = ((pw0, pb0), (pw1, pb1), (pw2, pb2), (pw3, pb3), (pw4, pb4))
    params = jnp.concatenate([
        jnp.concatenate([pw[8, :], pb]) for pw, pb in pws
    ])
    packed = jnp.concatenate([
        ((jax.lax.bitcast_convert_type(pw[0:8:2, :].astype(jnp.bfloat16),
                                       jnp.uint16).astype(jnp.uint32) << 16) |
         jax.lax.bitcast_convert_type(pw[1:8:2, :].astype(jnp.bfloat16),
                                      jnp.uint16).astype(jnp.uint32)
         ).reshape(-1)
        for pw, _ in pws
    ])

    preds = (pred0, pred1, pred2, pred3, pred4)
    in_specs = [
        pl.BlockSpec(memory_space=pltpu.SMEM),
        pl.BlockSpec(memory_space=pltpu.SMEM),
        pl.BlockSpec((1, 3, H + 2, W + 2), lambda n: (n, 0, 0, 0)),
        pl.BlockSpec((1, 1, H, W), lambda n: (n, 0, 0, 0)),
    ] + [
        pl.BlockSpec((1, 27, H // s, W // s), lambda n: (n, 0, 0, 0))
        for s in _POOLS
    ]
    out = pl.pallas_call(
        _loss_kernel,
        grid=(N,),
        in_specs=in_specs,
        out_specs=pl.BlockSpec((1, 8, 128), lambda n: (n, 0, 0)),
        out_shape=jax.ShapeDtypeStruct((N, 8, 128), jnp.float32),
        compiler_params=pltpu.CompilerParams(
            dimension_semantics=("parallel",)),
    )(params, packed, xp, mask, *preds)

    vals = jnp.sum(out[:, 0, :], axis=0)
    masksum = vals[5]
    loss = jnp.float32(0.0)
    for k, s in enumerate(_POOLS):
        loss = loss + vals[k] * (s * s) / masksum
    return loss


# speedup vs baseline: 81.1060x; 1.0485x over previous
import math

import jax
import jax.numpy as jnp
from jax.experimental import pallas as pl
import jax.experimental.pallas.tpu as pltpu

_POOLS = (16, 8, 4, 2, 1)
_NB = 9


def _tree(t, w):
    lo = jnp.where(t[2], jnp.where(t[3], w[3], w[2]),
                   jnp.where(t[1], w[1], w[0]))
    hi = jnp.where(t[6],
                   jnp.where(t[7], jnp.where(t[8], w[8], w[7]), w[6]),
                   jnp.where(t[5], w[5], w[4]))
    return jnp.where(t[4], hi, lo)


def _loss_kernel(params_ref, packed_ref, xp_ref, mask_ref, p0_ref, p1_ref,
                 p2_ref, p3_ref, p4_ref, out_ref):
    H = mask_ref.shape[2]
    W = mask_ref.shape[3]
    f32 = jnp.float32

    g0s = []
    g1s = []
    for c in range(3):
        x = xp_ref[0, c, :, :]
        up = jnp.concatenate([x[1:2, :], x[0:H - 1, :]], axis=0)
        dn = jnp.concatenate([x[1:H, :], x[H - 2:H - 1, :]], axis=0)
        vsum = up + 2.0 * x + dn
        lf = jnp.concatenate([x[:, 1:2], x[:, 0:W - 1]], axis=1)
        rt = jnp.concatenate([x[:, 1:W], x[:, W - 2:W - 1]], axis=1)
        hsum = lf + 2.0 * x + rt
        g0 = (jnp.concatenate([vsum[:, 1:2], vsum[:, 0:W - 1]], axis=1) -
              jnp.concatenate([vsum[:, 1:W], vsum[:, W - 2:W - 1]], axis=1))
        g1 = (jnp.concatenate([hsum[1:2, :], hsum[0:H - 1, :]], axis=0) -
              jnp.concatenate([hsum[1:H, :], hsum[H - 2:H - 1, :]], axis=0))
        g0s.append(g0)
        g1s.append(g1)

    mask2d = mask_ref[0, 0, :, :]
    masksum = jnp.sum(mask2d)

    cots = [1.0 / math.tan(k * math.pi / _NB) for k in range(1, _NB)]

    pred_refs = {16: p0_ref, 8: p1_ref, 4: p2_ref, 2: p3_ref, 1: p4_ref}
    nums = []
    for k, s in enumerate(_POOLS):
        Hs, Ws = H // s, W // s
        pred_ref = pred_refs[s]
        if s == 1:
            Mk = mask2d
        else:
            rp = (jax.lax.broadcasted_iota(jnp.int32, (Hs, H), 1) // s ==
                  jax.lax.broadcasted_iota(jnp.int32, (Hs, H), 0)).astype(f32)
            cp = (jax.lax.broadcasted_iota(jnp.int32, (W, Ws), 0) // s ==
                  jax.lax.broadcasted_iota(jnp.int32, (W, Ws), 1)).astype(f32)
            Mk = jnp.dot(jnp.dot(rp, mask2d, preferred_element_type=f32), cp,
                         preferred_element_type=f32) * (1.0 / (s * s))
            rs = (jax.lax.broadcasted_iota(jnp.int32, (Hs, H), 1) ==
                  s * jax.lax.broadcasted_iota(jnp.int32, (Hs, H), 0)
                  ).astype(f32)
            cs = (jax.lax.broadcasted_iota(jnp.int32, (W, Ws), 0) ==
                  s * jax.lax.broadcasted_iota(jnp.int32, (W, Ws), 1)
                  ).astype(f32)

        e = jnp.zeros((Hs, Ws), dtype=f32)
        for c in range(3):
            if s == 1:
                gg0, gg1 = g0s[c], g1s[c]
            else:
                gg0 = jnp.dot(jnp.dot(rs, g0s[c], preferred_element_type=f32),
                              cs, preferred_element_type=f32)
                gg1 = jnp.dot(jnp.dot(rs, g1s[c], preferred_element_type=f32),
                              cs, preferred_element_type=f32)
            mag = jnp.sqrt(gg0 * gg0 + gg1 * gg1)
            neg = gg0 < 0.0
            a0 = jnp.where(neg, -gg0, gg0)
            a1 = jnp.where(neg, -gg1, gg1)
            a1 = jnp.where(a0 == 0.0, jnp.abs(a1), a1)
            t = [None] + [a1 <= a0 * ck for ck in cots]
            base = k * (2 * _NB)
            baseu = k * (4 * _NB)
            hogs = [None] * _NB
            for j in range(4):
                w = [packed_ref[baseu + j * _NB + b] for b in range(_NB)]
                v = _tree(t, w)
                hi = jax.lax.bitcast_convert_type(
                    v & jnp.uint32(0xFFFF0000), jnp.float32)
                lo = jax.lax.bitcast_convert_type(v << 16, jnp.float32)
                hogs[2 * j] = params_ref[base + _NB + 2 * j] + mag * hi
                hogs[2 * j + 1] = params_ref[base + _NB + 2 * j + 1] + mag * lo
            w8 = [params_ref[base + b] for b in range(_NB)]
            hogs[8] = params_ref[base + _NB + 8] + mag * _tree(t, w8)
            nrm2 = hogs[0] * hogs[0]
            for o in range(1, _NB):
                nrm2 = nrm2 + hogs[o] * hogs[o]
            inv = 1.0 / jnp.maximum(jnp.sqrt(nrm2), 1e-12)
            for o in range(_NB):
                d = pred_ref[0, c * _NB + o, :, :] - hogs[o] * inv
                e = e + d * d
        nums.append(jnp.sum(e * Mk) * (1.0 / 27.0))

    lane = jax.lax.broadcasted_iota(jnp.int32, (8, 128), 1)
    tile = jnp.zeros((8, 128), dtype=f32)
    for k in range(5):
        tile = jnp.where(lane == k, nums[k], tile)
    tile = jnp.where(lane == 5, masksum, tile)
    out_ref[0, :, :] = tile


def kernel(pred0, pred1, pred2, pred3, pred4, imgs, mask, pw0, pb0, pw1, pb1,
           pw2, pb2, pw3, pb3, pw4, pb4):
    N, _, H, W = imgs.shape
    pws = ((pw0, pb0), (pw1, pb1), (pw2, pb2), (pw3, pb3), (pw4, pb4))
    params = jnp.concatenate([
        jnp.concatenate([pw[8, :], pb]) for pw, pb in pws
    ])
    packed = jnp.concatenate([
        ((jax.lax.bitcast_convert_type(pw[0:8:2, :].astype(jnp.bfloat16),
                                       jnp.uint16).astype(jnp.uint32) << 16) |
         jax.lax.bitcast_convert_type(pw[1:8:2, :].astype(jnp.bfloat16),
                                      jnp.uint16).astype(jnp.uint32)
         ).reshape(-1)
        for pw, _ in pws
    ])

    preds = (pred0, pred1, pred2, pred3, pred4)
    in_specs = [
        pl.BlockSpec(memory_space=pltpu.SMEM),
        pl.BlockSpec(memory_space=pltpu.SMEM),
        pl.BlockSpec((1, 3, H, W), lambda n: (n, 0, 0, 0)),
        pl.BlockSpec((1, 1, H, W), lambda n: (n, 0, 0, 0)),
    ] + [
        pl.BlockSpec((1, 27, H // s, W // s), lambda n: (n, 0, 0, 0))
        for s in _POOLS
    ]
    out = pl.pallas_call(
        _loss_kernel,
        grid=(N,),
        in_specs=in_specs,
        out_specs=pl.BlockSpec((1, 8, 128), lambda n: (n, 0, 0)),
        out_shape=jax.ShapeDtypeStruct((N, 8, 128), jnp.float32),
        compiler_params=pltpu.CompilerParams(
            dimension_semantics=("parallel",)),
    )(params, packed, imgs, mask, *preds)

    vals = jnp.sum(out[:, 0, :], axis=0)
    masksum = vals[5]
    loss = jnp.float32(0.0)
    for k, s in enumerate(_POOLS):
        loss = loss + vals[k] * (s * s) / masksum
    return loss


# speedup vs baseline: 82.1305x; 1.0126x over previous
import math

import jax
import jax.numpy as jnp
from jax.experimental import pallas as pl
import jax.experimental.pallas.tpu as pltpu

_POOLS = (16, 8, 4, 2, 1)
_NB = 9


def _tree(t, w):
    lo = jnp.where(t[2], jnp.where(t[3], w[3], w[2]),
                   jnp.where(t[1], w[1], w[0]))
    hi = jnp.where(t[6],
                   jnp.where(t[7], jnp.where(t[8], w[8], w[7]), w[6]),
                   jnp.where(t[5], w[5], w[4]))
    return jnp.where(t[4], hi, lo)


def _pack(hf, lf):
    hb = jax.lax.bitcast_convert_type(hf, jnp.uint32)
    lb = jax.lax.bitcast_convert_type(lf, jnp.uint32)
    return (hb & jnp.uint32(0xFFFF0000)) | (lb >> 16)


def _loss_kernel(pw0_ref, pb0_ref, pw1_ref, pb1_ref, pw2_ref, pb2_ref,
                 pw3_ref, pb3_ref, pw4_ref, pb4_ref, xp_ref, mask_ref,
                 p0_ref, p1_ref, p2_ref, p3_ref, p4_ref, out_ref):
    pwr = (pw0_ref, pw1_ref, pw2_ref, pw3_ref, pw4_ref)
    pbr = (pb0_ref, pb1_ref, pb2_ref, pb3_ref, pb4_ref)
    H = mask_ref.shape[2]
    W = mask_ref.shape[3]
    f32 = jnp.float32

    g0s = []
    g1s = []
    for c in range(3):
        x = xp_ref[0, c, :, :]
        up = jnp.concatenate([x[1:2, :], x[0:H - 1, :]], axis=0)
        dn = jnp.concatenate([x[1:H, :], x[H - 2:H - 1, :]], axis=0)
        vsum = up + 2.0 * x + dn
        lf = jnp.concatenate([x[:, 1:2], x[:, 0:W - 1]], axis=1)
        rt = jnp.concatenate([x[:, 1:W], x[:, W - 2:W - 1]], axis=1)
        hsum = lf + 2.0 * x + rt
        g0 = (jnp.concatenate([vsum[:, 1:2], vsum[:, 0:W - 1]], axis=1) -
              jnp.concatenate([vsum[:, 1:W], vsum[:, W - 2:W - 1]], axis=1))
        g1 = (jnp.concatenate([hsum[1:2, :], hsum[0:H - 1, :]], axis=0) -
              jnp.concatenate([hsum[1:H, :], hsum[H - 2:H - 1, :]], axis=0))
        g0s.append(g0)
        g1s.append(g1)

    mask2d = mask_ref[0, 0, :, :]
    masksum = jnp.sum(mask2d)

    cots = [1.0 / math.tan(k * math.pi / _NB) for k in range(1, _NB)]

    pred_refs = {16: p0_ref, 8: p1_ref, 4: p2_ref, 2: p3_ref, 1: p4_ref}
    nums = []
    for k, s in enumerate(_POOLS):
        Hs, Ws = H // s, W // s
        pred_ref = pred_refs[s]
        if s == 1:
            Mk = mask2d
        else:
            rp = (jax.lax.broadcasted_iota(jnp.int32, (Hs, H), 1) // s ==
                  jax.lax.broadcasted_iota(jnp.int32, (Hs, H), 0)).astype(f32)
            cp = (jax.lax.broadcasted_iota(jnp.int32, (W, Ws), 0) // s ==
                  jax.lax.broadcasted_iota(jnp.int32, (W, Ws), 1)).astype(f32)
            Mk = jnp.dot(jnp.dot(rp, mask2d, preferred_element_type=f32), cp,
                         preferred_element_type=f32) * (1.0 / (s * s))
            rs = (jax.lax.broadcasted_iota(jnp.int32, (Hs, H), 1) ==
                  s * jax.lax.broadcasted_iota(jnp.int32, (Hs, H), 0)
                  ).astype(f32)
            cs = (jax.lax.broadcasted_iota(jnp.int32, (W, Ws), 0) ==
                  s * jax.lax.broadcasted_iota(jnp.int32, (W, Ws), 1)
                  ).astype(f32)

        e = jnp.zeros((Hs, Ws), dtype=f32)
        for c in range(3):
            if s == 1:
                gg0, gg1 = g0s[c], g1s[c]
            else:
                gg0 = jnp.dot(jnp.dot(rs, g0s[c], preferred_element_type=f32),
                              cs, preferred_element_type=f32)
                gg1 = jnp.dot(jnp.dot(rs, g1s[c], preferred_element_type=f32),
                              cs, preferred_element_type=f32)
            mag = jnp.sqrt(gg0 * gg0 + gg1 * gg1)
            neg = gg0 < 0.0
            a0 = jnp.where(neg, -gg0, gg0)
            a1 = jnp.where(neg, -gg1, gg1)
            a1 = jnp.where(a0 == 0.0, jnp.abs(a1), a1)
            t = [None] + [a1 <= a0 * ck for ck in cots]
            hogs = [None] * _NB
            for j in range(4):
                w = [_pack(pwr[k][2 * j, b], pwr[k][2 * j + 1, b])
                     for b in range(_NB)]
                v = _tree(t, w)
                hi = jax.lax.bitcast_convert_type(
                    v & jnp.uint32(0xFFFF0000), jnp.float32)
                lo = jax.lax.bitcast_convert_type(v << 16, jnp.float32)
                hogs[2 * j] = pbr[k][2 * j] + mag * hi
                hogs[2 * j + 1] = pbr[k][2 * j + 1] + mag * lo
            w8 = [pwr[k][8, b] for b in range(_NB)]
            hogs[8] = pbr[k][8] + mag * _tree(t, w8)
            nrm2 = hogs[0] * hogs[0]
            for o in range(1, _NB):
                nrm2 = nrm2 + hogs[o] * hogs[o]
            inv = 1.0 / jnp.maximum(jnp.sqrt(nrm2), 1e-12)
            for o in range(_NB):
                d = pred_ref[0, c * _NB + o, :, :] - hogs[o] * inv
                e = e + d * d
        nums.append(jnp.sum(e * Mk) * (1.0 / 27.0))

    lane = jax.lax.broadcasted_iota(jnp.int32, (8, 128), 1)
    tile = jnp.zeros((8, 128), dtype=f32)
    for k in range(5):
        tile = jnp.where(lane == k, nums[k], tile)
    tile = jnp.where(lane == 5, masksum, tile)

    @pl.when(pl.program_id(0) == 0)
    def _():
        out_ref[:, :] = tile

    @pl.when(pl.program_id(0) != 0)
    def _():
        out_ref[:, :] = out_ref[:, :] + tile


def kernel(pred0, pred1, pred2, pred3, pred4, imgs, mask, pw0, pb0, pw1, pb1,
           pw2, pb2, pw3, pb3, pw4, pb4):
    N, _, H, W = imgs.shape
    preds = (pred0, pred1, pred2, pred3, pred4)
    in_specs = [pl.BlockSpec(memory_space=pltpu.SMEM)] * 10 + [
        pl.BlockSpec((1, 3, H, W), lambda n: (n, 0, 0, 0)),
        pl.BlockSpec((1, 1, H, W), lambda n: (n, 0, 0, 0)),
    ] + [
        pl.BlockSpec((1, 27, H // s, W // s), lambda n: (n, 0, 0, 0))
        for s in _POOLS
    ]
    out = pl.pallas_call(
        _loss_kernel,
        grid=(N,),
        in_specs=in_specs,
        out_specs=pl.BlockSpec((8, 128), lambda n: (0, 0)),
        out_shape=jax.ShapeDtypeStruct((8, 128), jnp.float32),
        compiler_params=pltpu.CompilerParams(
            dimension_semantics=("arbitrary",)),
    )(pw0, pb0, pw1, pb1, pw2, pb2, pw3, pb3, pw4, pb4, imgs, mask, *preds)

    vals = out[0, :]
    masksum = vals[5]
    loss = jnp.float32(0.0)
    for k, s in enumerate(_POOLS):
        loss = loss + vals[k] * (s * s) / masksum
    return loss


# speedup vs baseline: 83.3641x; 1.0150x over previous
import math

import jax
import jax.numpy as jnp
from jax.experimental import pallas as pl
import jax.experimental.pallas.tpu as pltpu

_POOLS = (16, 8, 4, 2, 1)
_NB = 9


def _tree(t, w):
    lo = jnp.where(t[2], jnp.where(t[3], w[3], w[2]),
                   jnp.where(t[1], w[1], w[0]))
    hi = jnp.where(t[6],
                   jnp.where(t[7], jnp.where(t[8], w[8], w[7]), w[6]),
                   jnp.where(t[5], w[5], w[4]))
    return jnp.where(t[4], hi, lo)


def _pack(hf, lf):
    hb = jax.lax.bitcast_convert_type(hf, jnp.uint32)
    lb = jax.lax.bitcast_convert_type(lf, jnp.uint32)
    return (hb & jnp.uint32(0xFFFF0000)) | (lb >> 16)


def _loss_kernel(pw0_ref, pb0_ref, pw1_ref, pb1_ref, pw2_ref, pb2_ref,
                 pw3_ref, pb3_ref, pw4_ref, pb4_ref, xp_ref, mask_ref,
                 p0_ref, p1_ref, p2_ref, p3_ref, p4_ref, out_ref):
    pwr = (pw0_ref, pw1_ref, pw2_ref, pw3_ref, pw4_ref)
    pbr = (pb0_ref, pb1_ref, pb2_ref, pb3_ref, pb4_ref)
    H = mask_ref.shape[2]
    W = mask_ref.shape[3]
    f32 = jnp.float32

    pool_mats = {}
    for s in _POOLS:
        if s == 1:
            continue
        Hs, Ws = H // s, W // s
        rp = (jax.lax.broadcasted_iota(jnp.int32, (Hs, H), 1) // s ==
              jax.lax.broadcasted_iota(jnp.int32, (Hs, H), 0)).astype(f32)
        cp = (jax.lax.broadcasted_iota(jnp.int32, (W, Ws), 0) // s ==
              jax.lax.broadcasted_iota(jnp.int32, (W, Ws), 1)).astype(f32)
        rs = (jax.lax.broadcasted_iota(jnp.int32, (Hs, H), 1) ==
              s * jax.lax.broadcasted_iota(jnp.int32, (Hs, H), 0)).astype(f32)
        cs = (jax.lax.broadcasted_iota(jnp.int32, (W, Ws), 0) ==
              s * jax.lax.broadcasted_iota(jnp.int32, (W, Ws), 1)).astype(f32)
        pool_mats[s] = (rp, cp, rs, cs)

    nums_acc = [jnp.float32(0.0)] * 5
    masksum_acc = jnp.float32(0.0)
    for img in range(xp_ref.shape[0]):
        nums, masksum = _one_image(pwr, pbr, xp_ref, mask_ref, p0_ref, p1_ref,
                                   p2_ref, p3_ref, p4_ref, img, pool_mats)
        for k in range(5):
            nums_acc[k] = nums_acc[k] + nums[k]
        masksum_acc = masksum_acc + masksum
    nums = nums_acc
    masksum = masksum_acc

    lane = jax.lax.broadcasted_iota(jnp.int32, (8, 128), 1)
    tile = jnp.zeros((8, 128), dtype=f32)
    for k in range(5):
        tile = jnp.where(lane == k, nums[k], tile)
    tile = jnp.where(lane == 5, masksum, tile)

    @pl.when(pl.program_id(0) == 0)
    def _():
        out_ref[:, :] = tile

    @pl.when(pl.program_id(0) != 0)
    def _():
        out_ref[:, :] = out_ref[:, :] + tile


def _one_image(pwr, pbr, xp_ref, mask_ref, p0_ref, p1_ref, p2_ref, p3_ref,
               p4_ref, img, pool_mats):
    H = mask_ref.shape[2]
    W = mask_ref.shape[3]
    f32 = jnp.float32
    g0s = []
    g1s = []
    for c in range(3):
        x = xp_ref[img, c, :, :]
        up = jnp.concatenate([x[1:2, :], x[0:H - 1, :]], axis=0)
        dn = jnp.concatenate([x[1:H, :], x[H - 2:H - 1, :]], axis=0)
        vsum = up + 2.0 * x + dn
        lf = jnp.concatenate([x[:, 1:2], x[:, 0:W - 1]], axis=1)
        rt = jnp.concatenate([x[:, 1:W], x[:, W - 2:W - 1]], axis=1)
        hsum = lf + 2.0 * x + rt
        g0 = (jnp.concatenate([vsum[:, 1:2], vsum[:, 0:W - 1]], axis=1) -
              jnp.concatenate([vsum[:, 1:W], vsum[:, W - 2:W - 1]], axis=1))
        g1 = (jnp.concatenate([hsum[1:2, :], hsum[0:H - 1, :]], axis=0) -
              jnp.concatenate([hsum[1:H, :], hsum[H - 2:H - 1, :]], axis=0))
        g0s.append(g0)
        g1s.append(g1)

    mask2d = mask_ref[img, 0, :, :]
    masksum = jnp.sum(mask2d)

    cots = [1.0 / math.tan(k * math.pi / _NB) for k in range(1, _NB)]

    pred_refs = {16: p0_ref, 8: p1_ref, 4: p2_ref, 2: p3_ref, 1: p4_ref}
    nums = []
    for k, s in enumerate(_POOLS):
        Hs, Ws = H // s, W // s
        pred_ref = pred_refs[s]
        if s == 1:
            Mk = mask2d
        else:
            rp, cp, rs, cs = pool_mats[s]
            Mk = jnp.dot(jnp.dot(rp, mask2d, preferred_element_type=f32), cp,
                         preferred_element_type=f32) * (1.0 / (s * s))

        e = jnp.zeros((Hs, Ws), dtype=f32)
        for c in range(3):
            if s == 1:
                gg0, gg1 = g0s[c], g1s[c]
            else:
                gg0 = jnp.dot(jnp.dot(rs, g0s[c], preferred_element_type=f32),
                              cs, preferred_element_type=f32)
                gg1 = jnp.dot(jnp.dot(rs, g1s[c], preferred_element_type=f32),
                              cs, preferred_element_type=f32)
            mag = jnp.sqrt(gg0 * gg0 + gg1 * gg1)
            neg = gg0 < 0.0
            a0 = jnp.where(neg, -gg0, gg0)
            a1 = jnp.where(neg, -gg1, gg1)
            a1 = jnp.where(a0 == 0.0, jnp.abs(a1), a1)
            t = [None] + [a1 <= a0 * ck for ck in cots]
            hogs = [None] * _NB
            for j in range(4):
                w = [_pack(pwr[k][2 * j, b], pwr[k][2 * j + 1, b])
                     for b in range(_NB)]
                v = _tree(t, w)
                hi = jax.lax.bitcast_convert_type(
                    v & jnp.uint32(0xFFFF0000), jnp.float32)
                lo = jax.lax.bitcast_convert_type(v << 16, jnp.float32)
                hogs[2 * j] = pbr[k][2 * j] + mag * hi
                hogs[2 * j + 1] = pbr[k][2 * j + 1] + mag * lo
            w8 = [pwr[k][8, b] for b in range(_NB)]
            hogs[8] = pbr[k][8] + mag * _tree(t, w8)
            nrm2 = hogs[0] * hogs[0]
            for o in range(1, _NB):
                nrm2 = nrm2 + hogs[o] * hogs[o]
            inv = 1.0 / jnp.maximum(jnp.sqrt(nrm2), 1e-12)
            for o in range(_NB):
                d = pred_ref[img, c * _NB + o, :, :] - hogs[o] * inv
                e = e + d * d
        nums.append(jnp.sum(e * Mk) * (1.0 / 27.0))
    return nums, masksum


def kernel(pred0, pred1, pred2, pred3, pred4, imgs, mask, pw0, pb0, pw1, pb1,
           pw2, pb2, pw3, pb3, pw4, pb4):
    N, _, H, W = imgs.shape
    preds = (pred0, pred1, pred2, pred3, pred4)
    G = 2 if N % 2 == 0 else 1
    in_specs = [pl.BlockSpec(memory_space=pltpu.SMEM)] * 10 + [
        pl.BlockSpec((G, 3, H, W), lambda n: (n, 0, 0, 0)),
        pl.BlockSpec((G, 1, H, W), lambda n: (n, 0, 0, 0)),
    ] + [
        pl.BlockSpec((G, 27, H // s, W // s), lambda n: (n, 0, 0, 0))
        for s in _POOLS
    ]
    out = pl.pallas_call(
        _loss_kernel,
        grid=(N // G,),
        in_specs=in_specs,
        out_specs=pl.BlockSpec((8, 128), lambda n: (0, 0)),
        out_shape=jax.ShapeDtypeStruct((8, 128), jnp.float32),
        compiler_params=pltpu.CompilerParams(
            dimension_semantics=("arbitrary",)),
    )(pw0, pb0, pw1, pb1, pw2, pb2, pw3, pb3, pw4, pb4, imgs, mask, *preds)

    vals = out[0, :]
    masksum = vals[5]
    loss = jnp.float32(0.0)
    for k, s in enumerate(_POOLS):
        loss = loss + vals[k] * (s * s) / masksum
    return loss
